# Initial kernel scaffold; baseline (speedup 1.0000x reference)
import jax
import jax.numpy as jnp
from jax.experimental import pallas as pl


def kernel(inputs, target, emb_table, W, b):
    raise NotImplementedError("write your pallas kernel here")


# speedup vs baseline: 4.0575x; 4.0575x over previous
import functools

import jax
import jax.numpy as jnp
from jax import lax
from jax.experimental import pallas as pl
from jax.experimental.pallas import tpu as pltpu
from jax.experimental.pallas import tpu_sc as plsc

VOCAB = 100000
EMBED = 64
CTX = 20
AW = 128
VPAD = 100352
BB = 512
BV = 2048
NEG = -1e30


def _sc_gather(emb_pad, idx_flat, a_table, tgt_idx):
    n_emb = idx_flat.shape[0]
    n_tgt = tgt_idx.shape[0]
    info = plsc.get_sparse_core_info()
    nw = info.num_cores * info.num_subcores
    CH = 128
    emb_per_w = n_emb // nw
    tgt_per_w = n_tgt // nw
    n_chunks = emb_per_w // CH

    mesh = plsc.VectorSubcoreMesh(core_axis_name="c", subcore_axis_name="s")

    @functools.partial(
        pl.kernel,
        mesh=mesh,
        out_type=[
            jax.ShapeDtypeStruct((n_emb, AW), jnp.float32),
            jax.ShapeDtypeStruct((n_tgt, AW), jnp.float32),
        ],
        scratch_types=[
            pltpu.VMEM((CH,), jnp.int32),
            pltpu.VMEM((CH, AW), jnp.float32),
            pltpu.VMEM((tgt_per_w,), jnp.int32),
            pltpu.VMEM((tgt_per_w, AW), jnp.float32),
            pltpu.SemaphoreType.DMA,
        ],
    )
    def k(emb_hbm, idx_hbm, a_hbm, t_hbm, g_out, tg_out,
          idx_v, rows_v, tidx_v, trows_v, sem):
        wid = lax.axis_index("s") * info.num_cores + lax.axis_index("c")
        tbase = wid * tgt_per_w
        pltpu.sync_copy(t_hbm.at[pl.ds(tbase, tgt_per_w)], tidx_v)
        pltpu.async_copy(a_hbm.at[tidx_v], trows_v, sem).wait()
        pltpu.sync_copy(trows_v, tg_out.at[pl.ds(tbase, tgt_per_w)])
        for c in range(n_chunks):
            base = wid * emb_per_w + c * CH
            pltpu.sync_copy(idx_hbm.at[pl.ds(base, CH)], idx_v)
            pltpu.async_copy(emb_hbm.at[idx_v], rows_v, sem).wait()
            pltpu.sync_copy(rows_v, g_out.at[pl.ds(base, CH)])

    return k(emb_pad, idx_flat, a_table, tgt_idx)


def _tca_body(g_ref, tg_ref, s_ref, tgt_ref):
    s = g_ref[:, 0:AW]
    for j in range(1, CTX):
        s = s + g_ref[:, j * AW:(j + 1) * AW]
    s_aug = jnp.concatenate(
        [s[:, 0:EMBED], jnp.ones((BB, 1), jnp.float32),
         jnp.zeros((BB, AW - EMBED - 1), jnp.float32)], axis=1)
    s_ref[...] = s_aug.astype(jnp.bfloat16)
    tgt_ref[...] = jnp.sum(s_aug * tg_ref[...], axis=1, keepdims=True)


def _tcb_body(s_ref, at_ref, tgt_ref, out_ref, acc_ref):
    iv = pl.program_id(0)
    nv = pl.num_programs(0)
    nb_tot = s_ref.shape[0]

    @pl.when(iv == 0)
    def _init():
        acc_ref[...] = jnp.zeros_like(acc_ref)

    logits = lax.dot_general(
        s_ref[...], at_ref[...], (((1,), (0,)), ((), ())),
        preferred_element_type=jnp.float32)
    acc_ref[...] += jnp.sum(jnp.exp(logits), axis=1, keepdims=True)

    @pl.when(iv == nv - 1)
    def _fin():
        nll = jnp.log(acc_ref[...]) - tgt_ref[...]
        out_ref[...] = jnp.full((1, 1), jnp.sum(nll) / nb_tot, jnp.float32)


def _tc_loss(g2, at_bf, tg):
    b = g2.shape[0]
    nb = b // BB
    s_bf, tgt = pl.pallas_call(
        _tca_body,
        grid=(nb,),
        in_specs=[
            pl.BlockSpec((BB, CTX * AW), lambda i: (i, 0)),
            pl.BlockSpec((BB, AW), lambda i: (i, 0)),
        ],
        out_specs=[
            pl.BlockSpec((BB, AW), lambda i: (i, 0)),
            pl.BlockSpec((BB, 1), lambda i: (i, 0)),
        ],
        out_shape=[
            jax.ShapeDtypeStruct((b, AW), jnp.bfloat16),
            jax.ShapeDtypeStruct((b, 1), jnp.float32),
        ],
    )(g2, tg)
    nv = VPAD // BV
    out = pl.pallas_call(
        _tcb_body,
        grid=(nv,),
        in_specs=[
            pl.BlockSpec((b, AW), lambda i: (0, 0)),
            pl.BlockSpec((AW, BV), lambda i: (0, i)),
            pl.BlockSpec((b, 1), lambda i: (0, 0)),
        ],
        out_specs=pl.BlockSpec((1, 1), lambda i: (0, 0)),
        out_shape=jax.ShapeDtypeStruct((1, 1), jnp.float32),
        scratch_shapes=[pltpu.VMEM((b, 1), jnp.float32)],
    )(s_bf, at_bf, tgt)
    return out[0, 0]


def kernel(inputs, target, emb_table, W, b):
    bsz, ctx = inputs.shape
    v, e = emb_table.shape
    idx_flat = inputs.reshape(-1).astype(jnp.int32)
    tgt_idx = target.astype(jnp.int32)
    emb_pad = jnp.concatenate(
        [emb_table, jnp.zeros((v, AW - e), jnp.float32)], axis=1)
    a_table = jnp.concatenate(
        [W, b[:, None], jnp.zeros((v, AW - e - 1), jnp.float32)], axis=1)
    g, tg = _sc_gather(emb_pad, idx_flat, a_table, tgt_idx)
    g2 = g.reshape(bsz, ctx * AW)
    pad_cols = jnp.zeros((AW, VPAD - v), jnp.float32).at[e, :].set(NEG)
    at_t = jnp.concatenate(
        [W.T, b[None, :], jnp.zeros((AW - e - 1, v), jnp.float32)], axis=0)
    at_t = jnp.concatenate([at_t, pad_cols], axis=1).astype(jnp.bfloat16)
    return _tc_loss(g2, at_t, tg)


# speedup vs baseline: 4.3325x; 1.0678x over previous
import functools

import jax
import jax.numpy as jnp
from jax import lax
from jax.experimental import pallas as pl
from jax.experimental.pallas import tpu as pltpu
from jax.experimental.pallas import tpu_sc as plsc

VOCAB = 100000
EMBED = 64
CTX = 20
AW = 128
AK = 80
VPAD = 100352
BB = 512
BV = 2048
NEG = -1e30
NBUF = 4


def _sc_gather(emb_pad, idx2, a_table, tgt_idx):
    n_idx = idx2.shape[0]
    chw = 128
    n_ch = n_idx // chw
    n_emb = n_ch * chw
    n_tgt = tgt_idx.shape[0]
    info = plsc.get_sparse_core_info()
    nw = info.num_cores * info.num_subcores
    ch_per_w = n_ch // nw
    tgt_per_w = n_tgt // nw

    mesh = plsc.VectorSubcoreMesh(core_axis_name="c", subcore_axis_name="s")

    @functools.partial(
        pl.kernel,
        mesh=mesh,
        out_type=[
            jax.ShapeDtypeStruct((n_emb, AW), jnp.float32),
            jax.ShapeDtypeStruct((n_tgt, AW), jnp.float32),
        ],
        scratch_types=[
            pltpu.VMEM((ch_per_w * chw,), jnp.int32),
            pltpu.VMEM((NBUF, chw, AW), jnp.float32),
            pltpu.VMEM((tgt_per_w,), jnp.int32),
            pltpu.VMEM((tgt_per_w, AW), jnp.float32),
            pltpu.SemaphoreType.DMA,
            pltpu.SemaphoreType.DMA,
        ],
    )
    def k(emb_hbm, idx_hbm, a_hbm, t_hbm, g_out, tg_out,
          idx_buf, bufs, tidx_v, trows_v, gs, ws):
        wid = lax.axis_index("s") * info.num_cores + lax.axis_index("c")
        pltpu.sync_copy(idx_hbm.at[pl.ds(wid * ch_per_w * chw, ch_per_w * chw)], idx_buf)
        tbase = wid * tgt_per_w
        pltpu.sync_copy(t_hbm.at[pl.ds(tbase, tgt_per_w)], tidx_v)
        th = pltpu.async_copy(a_hbm.at[tidx_v], trows_v, gs)
        gh = [None] * ch_per_w
        wh = [None] * ch_per_w
        for c in range(NBUF):
            gh[c] = pltpu.async_copy(
                emb_hbm.at[idx_buf.at[pl.ds(c * chw, chw)]], bufs.at[c], gs)
        th.wait()
        twh = pltpu.async_copy(trows_v, tg_out.at[pl.ds(tbase, tgt_per_w)], ws)
        for c in range(ch_per_w):
            gh[c].wait()
            wh[c] = pltpu.async_copy(
                bufs.at[c % NBUF],
                g_out.at[pl.ds(wid * ch_per_w * chw + c * chw, chw)], ws)
            nxt = c + NBUF
            if nxt < ch_per_w:
                wh[c].wait()
                gh[nxt] = pltpu.async_copy(
                    emb_hbm.at[idx_buf.at[pl.ds(nxt * chw, chw)]],
                    bufs.at[nxt % NBUF], gs)
        twh.wait()
        for c in range(ch_per_w - NBUF, ch_per_w):
            wh[c].wait()

    return k(emb_pad, idx2, a_table, tgt_idx)


def _tca_body(g_ref, tg_ref, s_ref, tgt_ref):
    s = g_ref[:, 0:AW]
    for j in range(1, CTX):
        s = s + g_ref[:, j * AW:(j + 1) * AW]
    s_aug = jnp.concatenate(
        [s[:, 0:EMBED], jnp.ones((BB, 1), jnp.float32),
         jnp.zeros((BB, AK - EMBED - 1), jnp.float32)], axis=1)
    s_ref[...] = s_aug.astype(jnp.bfloat16)
    tgt_ref[...] = jnp.sum(s_aug * tg_ref[:, 0:AK], axis=1, keepdims=True)


def _tcb_body(s_ref, at_ref, tgt_ref, out_ref, acc_ref):
    iv = pl.program_id(0)
    nv = pl.num_programs(0)
    nb_tot = s_ref.shape[0]

    @pl.when(iv == 0)
    def _init():
        acc_ref[...] = jnp.zeros_like(acc_ref)

    logits = lax.dot_general(
        s_ref[...], at_ref[...], (((1,), (0,)), ((), ())),
        preferred_element_type=jnp.float32)
    acc_ref[...] += jnp.sum(jnp.exp(logits), axis=1, keepdims=True)

    @pl.when(iv == nv - 1)
    def _fin():
        nll = jnp.log(acc_ref[...]) - tgt_ref[...]
        out_ref[...] = jnp.full((1, 1), jnp.sum(nll) / nb_tot, jnp.float32)


def _tc_loss(g2, at_bf, tg):
    b = g2.shape[0]
    nb = b // BB
    s_bf, tgt = pl.pallas_call(
        _tca_body,
        grid=(nb,),
        in_specs=[
            pl.BlockSpec((BB, CTX * AW), lambda i: (i, 0)),
            pl.BlockSpec((BB, AW), lambda i: (i, 0)),
        ],
        out_specs=[
            pl.BlockSpec((BB, AK), lambda i: (i, 0)),
            pl.BlockSpec((BB, 1), lambda i: (i, 0)),
        ],
        out_shape=[
            jax.ShapeDtypeStruct((b, AK), jnp.bfloat16),
            jax.ShapeDtypeStruct((b, 1), jnp.float32),
        ],
    )(g2, tg)
    nv = VPAD // BV
    out = pl.pallas_call(
        _tcb_body,
        grid=(nv,),
        in_specs=[
            pl.BlockSpec((b, AK), lambda i: (0, 0)),
            pl.BlockSpec((AK, BV), lambda i: (0, i)),
            pl.BlockSpec((b, 1), lambda i: (0, 0)),
        ],
        out_specs=pl.BlockSpec((1, 1), lambda i: (0, 0)),
        out_shape=jax.ShapeDtypeStruct((1, 1), jnp.float32),
        scratch_shapes=[pltpu.VMEM((b, 1), jnp.float32)],
    )(s_bf, at_bf, tgt)
    return out[0, 0]


def kernel(inputs, target, emb_table, W, b):
    bsz, ctx = inputs.shape
    v, e = emb_table.shape
    idx2 = inputs.reshape(-1).astype(jnp.int32)
    tgt_idx = target.astype(jnp.int32)
    emb_pad = jnp.concatenate(
        [emb_table, jnp.zeros((v, AW - e), jnp.float32)], axis=1)
    a_table = jnp.concatenate(
        [W, b[:, None], jnp.zeros((v, AW - e - 1), jnp.float32)], axis=1)
    g, tg = _sc_gather(emb_pad, idx2, a_table, tgt_idx)
    g2 = g.reshape(bsz, ctx * AW)
    pad_cols = jnp.zeros((AK, VPAD - v), jnp.float32).at[e, :].set(NEG)
    at_t = jnp.concatenate(
        [W.T, b[None, :], jnp.zeros((AK - e - 1, v), jnp.float32)], axis=0)
    at_t = jnp.concatenate([at_t, pad_cols], axis=1).astype(jnp.bfloat16)
    return _tc_loss(g2, at_t, tg)


# speedup vs baseline: 5.5034x; 1.2702x over previous
import functools

import jax
import jax.numpy as jnp
from jax import lax
from jax.experimental import pallas as pl
from jax.experimental.pallas import tpu as pltpu
from jax.experimental.pallas import tpu_sc as plsc

VOCAB = 100000
EMBED = 64
CTX = 20
AW = 128
AK = 80
VPAD = 100352
BV = 2048
NEG = -1e30
INFLIGHT = 8


def _sc_gather(t_table, idx_t, tgt_idx, tb_idx, bp, zeros_blk):
    n_idx = idx_t.shape[0]
    n_tgt = tgt_idx.shape[0]
    info = plsc.get_sparse_core_info()
    nw = info.num_cores * info.num_subcores
    chw = 128
    ch_per_w = n_idx // (nw * chw)
    tgt_per_w = n_tgt // nw

    mesh = plsc.VectorSubcoreMesh(core_axis_name="c", subcore_axis_name="s")

    @functools.partial(
        pl.kernel,
        mesh=mesh,
        out_type=[
            jax.ShapeDtypeStruct((n_tgt, AW), jnp.float32),
            jax.ShapeDtypeStruct((n_tgt, AW), jnp.float32),
            jax.ShapeDtypeStruct((n_tgt, AW), jnp.float32),
        ],
        scratch_types=[
            pltpu.VMEM((ch_per_w * chw,), jnp.int32),
            pltpu.VMEM((chw, AW), jnp.float32),
            pltpu.VMEM((tgt_per_w,), jnp.int32),
            pltpu.VMEM((tgt_per_w,), jnp.int32),
            pltpu.VMEM((tgt_per_w, AW), jnp.float32),
            pltpu.VMEM((tgt_per_w, AW), jnp.float32),
            pltpu.SemaphoreType.DMA,
            pltpu.SemaphoreType.DMA,
            pltpu.SemaphoreType.DMA,
        ],
    )
    def k(t_hbm, idx_hbm, tgt_hbm, tbi_hbm, bp_hbm, z_hbm,
          s_out, tg_out, bg_out,
          idx_all, acc, tidx, tbidx, trows, brows, gs, ts, ws):
        wid = lax.axis_index("s") * info.num_cores + lax.axis_index("c")
        base = wid * tgt_per_w
        pltpu.sync_copy(
            idx_hbm.at[pl.ds(wid * ch_per_w * chw, ch_per_w * chw)], idx_all)
        pltpu.sync_copy(tgt_hbm.at[pl.ds(base, tgt_per_w)], tidx)
        pltpu.sync_copy(tbi_hbm.at[pl.ds(base, tgt_per_w)], tbidx)
        th = pltpu.async_copy(t_hbm.at[tidx], trows, ts)
        bh = pltpu.async_copy(bp_hbm.at[tbidx], brows, ts)
        pltpu.sync_copy(z_hbm, acc)
        gh = [None] * ch_per_w
        for j in range(ch_per_w):
            if j >= INFLIGHT:
                gh[j - INFLIGHT].wait()
            gh[j] = pltpu.async_copy(
                t_hbm.at[idx_all.at[pl.ds(j * chw, chw)]], acc, gs, add=True)
        for j in range(ch_per_w - INFLIGHT, ch_per_w):
            gh[j].wait()
        w0 = pltpu.async_copy(acc, s_out.at[pl.ds(base, tgt_per_w)], ws)
        th.wait()
        w1 = pltpu.async_copy(trows, tg_out.at[pl.ds(base, tgt_per_w)], ws)
        bh.wait()
        w2 = pltpu.async_copy(brows, bg_out.at[pl.ds(base, tgt_per_w)], ws)
        w0.wait()
        w1.wait()
        w2.wait()

    return k(t_table, idx_t, tgt_idx, tb_idx, bp, zeros_blk)


def _tc_body(s_ref, tg_ref, bg_ref, oh_ref, at_ref, out_ref,
             sbf_ref, tgt_ref, acc_ref):
    iv = pl.program_id(0)
    nv = pl.num_programs(0)
    nb_tot = s_ref.shape[0]

    @pl.when(iv == 0)
    def _init():
        s = s_ref[:, 0:EMBED]
        s_aug = jnp.concatenate(
            [s, jnp.ones((nb_tot, 1), jnp.float32),
             jnp.zeros((nb_tot, AK - EMBED - 1), jnp.float32)], axis=1)
        sbf_ref[...] = s_aug.astype(jnp.bfloat16)
        tgt_ref[...] = (
            jnp.sum(s * tg_ref[:, EMBED:AW], axis=1, keepdims=True)
            + jnp.sum(bg_ref[...] * oh_ref[...], axis=1, keepdims=True))
        acc_ref[...] = jnp.zeros_like(acc_ref)

    logits = lax.dot_general(
        sbf_ref[...], at_ref[...], (((1,), (0,)), ((), ())),
        preferred_element_type=jnp.float32)
    acc_ref[...] += jnp.sum(jnp.exp(logits), axis=1, keepdims=True)

    @pl.when(iv == nv - 1)
    def _fin():
        nll = jnp.log(acc_ref[...]) - tgt_ref[...]
        out_ref[...] = jnp.full((1, 1), jnp.sum(nll) / nb_tot, jnp.float32)


def _tc_loss(s, tg, bg, oh, at_bf):
    b = s.shape[0]
    nv = VPAD // BV
    full = lambda i: (0, 0)
    out = pl.pallas_call(
        _tc_body,
        grid=(nv,),
        in_specs=[
            pl.BlockSpec((b, AW), full),
            pl.BlockSpec((b, AW), full),
            pl.BlockSpec((b, AW), full),
            pl.BlockSpec((b, AW), full),
            pl.BlockSpec((AK, BV), lambda i: (0, i)),
        ],
        out_specs=pl.BlockSpec((1, 1), full),
        out_shape=jax.ShapeDtypeStruct((1, 1), jnp.float32),
        scratch_shapes=[
            pltpu.VMEM((b, AK), jnp.bfloat16),
            pltpu.VMEM((b, 1), jnp.float32),
            pltpu.VMEM((b, 1), jnp.float32),
        ],
    )(s, tg, bg, oh, at_bf)
    return out[0, 0]


def kernel(inputs, target, emb_table, W, b):
    bsz, ctx = inputs.shape
    v, e = emb_table.shape
    nw = 32
    per_w = bsz // nw
    idx_t = (inputs.astype(jnp.int32)
             .reshape(nw, per_w, ctx).transpose(0, 2, 1).reshape(-1))
    tgt_idx = target.astype(jnp.int32)
    tb_idx = tgt_idx // AW
    oh = jax.nn.one_hot(tgt_idx % AW, AW, dtype=jnp.float32)
    t_table = jnp.concatenate([emb_table, W], axis=1)
    bp = jnp.concatenate(
        [b, jnp.zeros((-v) % AW, jnp.float32)]).reshape(-1, AW)
    zeros_blk = jnp.zeros((per_w, AW), jnp.float32)
    s, tg, bg = _sc_gather(t_table, idx_t, tgt_idx, tb_idx, bp, zeros_blk)
    pad_cols = jnp.zeros((AK, VPAD - v), jnp.float32).at[e, :].set(NEG)
    at_t = jnp.concatenate(
        [W.T, b[None, :], jnp.zeros((AK - e - 1, v), jnp.float32)], axis=0)
    at_t = jnp.concatenate([at_t, pad_cols], axis=1).astype(jnp.bfloat16)
    return _tc_loss(s, tg, bg, oh, at_t)
